# baseline (speedup 1.0000x reference)
import functools
import math

import jax
import jax.numpy as jnp
from jax import lax
from jax.experimental import pallas as pl
from jax.experimental.pallas import tpu as pltpu
from jax.experimental.pallas import tpu_sc as plsc

NODE_DIM = 128
TIME_DIM = 128
MODEL_DIM = NODE_DIM + TIME_DIM
N_HEAD = 4
HEAD_DIM = MODEL_DIM // N_HEAD
NNB = 20
_NW = 32


def _sc_gather(idx, tables):
    B = idx.shape[0]
    bpw = B // _NW
    C = min(bpw, 128)
    nch = bpw // C
    idx3 = idx.reshape(_NW, nch, C)
    nt = len(tables)
    mesh = plsc.VectorSubcoreMesh(core_axis_name="c", subcore_axis_name="s")
    out_type = tuple(
        jax.ShapeDtypeStruct((B, t.shape[1]), t.dtype) for t in tables
    )
    scratch = [pltpu.VMEM((nch, C), jnp.int32)]
    scratch += [pltpu.VMEM((C, t.shape[1]), t.dtype) for t in tables]
    scratch.append(pltpu.SemaphoreType.DMA)

    def body(*refs):
        tabs = refs[:nt]
        idx_ref = refs[nt]
        outs = refs[nt + 1:2 * nt + 1]
        idx_v = refs[2 * nt + 1]
        rows = refs[2 * nt + 2:2 * nt + 2 + nt]
        sem = refs[-1]
        wid = lax.axis_index("s") * 2 + lax.axis_index("c")
        pltpu.sync_copy(idx_ref.at[wid], idx_v)

        def step(i, carry):
            row0 = pl.multiple_of((wid * nch + i) * C, C)
            for t in range(nt):
                pltpu.async_copy(tabs[t].at[idx_v.at[i]], rows[t], sem).wait()
                pltpu.sync_copy(rows[t], outs[t].at[pl.ds(row0, C)])
            return carry

        lax.fori_loop(0, nch, step, 0)

    fn = pl.kernel(body, out_type=out_type, mesh=mesh, scratch_types=scratch)
    res = fn(*tables, idx3)
    return list(res)


def _attn_body(freq_r, eq_r, eng_r, tq_r, tnb_r, nb_r, wq_r, wk_r, wv_r,
               wo_r, bq_r, bk_r, bv_r, bo_r, mw1_r, mb1_r, mw2_r, mb2_r,
               out_r, *, QB):
    f32 = jnp.float32
    eq = eq_r[...]
    eng = eng_r[...]
    wq = wq_r[...]
    wk = wk_r[...]
    wv = wv_r[...]
    freq = freq_r[...]
    bf = jnp.bfloat16
    engb = eng.astype(bf)
    cq = jnp.sum(wq[NODE_DIM:, :], axis=0, keepdims=True) + bq_r[...]
    q = jnp.dot(eq.astype(bf), wq[:NODE_DIM, :].astype(bf),
                preferred_element_type=f32) + cq
    delta = tq_r[...] - tnb_r[...]
    ph = delta[:, :, None] * freq.reshape(1, 1, TIME_DIM)
    C = jnp.cos(ph).reshape(QB * NNB, TIME_DIM).astype(bf)
    k = (jnp.dot(engb, wk[:NODE_DIM, :].astype(bf),
                 preferred_element_type=f32)
         + jnp.dot(C, wk[NODE_DIM:, :].astype(bf),
                   preferred_element_type=f32)
         + bk_r[...])
    v = (jnp.dot(engb, wv[:NODE_DIM, :].astype(bf),
                 preferred_element_type=f32)
         + jnp.dot(C, wv[NODE_DIM:, :].astype(bf),
                   preferred_element_type=f32)
         + bv_r[...])
    qb = jnp.broadcast_to(
        q[:, None, :], (QB, NNB, MODEL_DIM)).reshape(QB * NNB, MODEL_DIM)
    di = lax.broadcasted_iota(jnp.int32, (MODEL_DIM, N_HEAD), 0)
    hi = lax.broadcasted_iota(jnp.int32, (MODEL_DIM, N_HEAD), 1)
    R = (di // HEAD_DIM == hi).astype(bf)
    pen = jnp.where(nb_r[...] == 0, -1e10, 0.0).astype(f32)
    S = (jnp.dot((qb * k).astype(bf), R, preferred_element_type=f32).reshape(
        QB, NNB, N_HEAD) * (1.0 / 8.0) + pen[:, :, None])
    m = jnp.max(S, axis=1, keepdims=True)
    Ex = jnp.exp(S - m)
    A = Ex / jnp.sum(Ex, axis=1, keepdims=True)
    Rt = (lax.broadcasted_iota(jnp.int32, (N_HEAD, MODEL_DIM), 1) // HEAD_DIM
          == lax.broadcasted_iota(jnp.int32, (N_HEAD, MODEL_DIM), 0)
          ).astype(bf)
    A4 = jnp.dot(A.reshape(QB * NNB, N_HEAD).astype(bf), Rt,
                 preferred_element_type=f32)
    o = jnp.sum((A4 * v).reshape(QB, NNB, MODEL_DIM), axis=1)
    o = (jnp.dot(o.astype(bf), wo_r[...].astype(bf),
                 preferred_element_type=f32) + bo_r[...])
    mw1 = mw1_r[...]
    g = (jnp.dot(o.astype(bf), mw1[:MODEL_DIM, :].astype(bf),
                 preferred_element_type=f32)
         + jnp.dot(eq.astype(bf), mw1[MODEL_DIM:, :].astype(bf),
                   preferred_element_type=f32)
         + mb1_r[...])
    g = jnp.maximum(g, 0.0)
    out_r[...] = (jnp.dot(g.astype(bf), mw2_r[...].astype(bf),
                          preferred_element_type=f32) + mb2_r[...])


def _attn_stage(freq, Eq, Eng, tq, Tnb, Nb, wq, wk, wv, wo, bq, bk, bv, bo,
                mw1, mb1, mw2, mb2, QB=128):
    Q = Eq.shape[0]
    grid = (Q // QB,)
    blk = lambda i: (i, 0)
    fix = lambda i: (0, 0)
    in_specs = [
        pl.BlockSpec((1, TIME_DIM), fix),
        pl.BlockSpec((QB, NODE_DIM), blk),
        pl.BlockSpec((QB * NNB, NODE_DIM), blk),
        pl.BlockSpec((QB, 1), blk),
        pl.BlockSpec((QB, NNB), blk),
        pl.BlockSpec((QB, NNB), blk),
        pl.BlockSpec((MODEL_DIM, MODEL_DIM), fix),
        pl.BlockSpec((MODEL_DIM, MODEL_DIM), fix),
        pl.BlockSpec((MODEL_DIM, MODEL_DIM), fix),
        pl.BlockSpec((MODEL_DIM, MODEL_DIM), fix),
        pl.BlockSpec((1, MODEL_DIM), fix),
        pl.BlockSpec((1, MODEL_DIM), fix),
        pl.BlockSpec((1, MODEL_DIM), fix),
        pl.BlockSpec((1, MODEL_DIM), fix),
        pl.BlockSpec((MODEL_DIM + NODE_DIM, NODE_DIM), fix),
        pl.BlockSpec((1, NODE_DIM), fix),
        pl.BlockSpec((NODE_DIM, NODE_DIM), fix),
        pl.BlockSpec((1, NODE_DIM), fix),
    ]
    return pl.pallas_call(
        functools.partial(_attn_body, QB=QB),
        grid=grid,
        in_specs=in_specs,
        out_specs=pl.BlockSpec((QB, NODE_DIM), blk),
        out_shape=jax.ShapeDtypeStruct((Q, NODE_DIM), jnp.float32),
    )(freq, Eq, Eng, tq, Tnb, Nb, wq, wk, wv, wo, bq, bk, bv, bo,
      mw1, mb1, mw2, mb2)


def _final_body(s_r, t_r, aw1_r, ab1_r, aw2_r, ab2_r, out_r):
    f32 = jnp.float32
    aw1 = aw1_r[...]
    h = (jnp.dot(s_r[...], aw1[:NODE_DIM, :], preferred_element_type=f32)
         + jnp.dot(t_r[...], aw1[NODE_DIM:, :], preferred_element_type=f32)
         + ab1_r[...])
    h = jnp.maximum(h, 0.0)
    out_r[...] = jnp.sum(h * aw2_r[...], axis=1, keepdims=True) + ab2_r[...]


def _final_stage(src, tgt, aw1, ab1, aw2t, ab2):
    B = src.shape[0]
    return pl.pallas_call(
        _final_body,
        out_shape=jax.ShapeDtypeStruct((B, 1), jnp.float32),
    )(src, tgt, aw1, ab1, aw2t, ab2)


def kernel(src_idx_l, target_idx_l, cut_time_l, adj_nodes, adj_times, emb,
           Wq, Wk, Wv, Wo, bq, bk, bv, bo, mW1, mb1, mW2, mb2,
           aW1, ab1, aW2, ab2):
    idx = jnp.concatenate([src_idx_l, target_idx_l]).astype(jnp.int32)
    cut = jnp.concatenate([cut_time_l, cut_time_l])
    B = idx.shape[0]
    adj_i = adj_nodes.astype(jnp.int32)

    zpad = jnp.zeros((adj_i.shape[0], 128 - 2 * NNB), jnp.int32)
    packed = jnp.concatenate(
        [adj_i, lax.bitcast_convert_type(adj_times, jnp.int32), zpad], axis=1)

    P1, E0 = _sc_gather(idx, [packed, emb])
    A1 = P1[:, :NNB]
    T1 = lax.bitcast_convert_type(P1[:, NNB:2 * NNB], jnp.float32)
    n1 = A1.reshape(-1)
    P2, E1 = _sc_gather(n1, [packed, emb])
    A2 = P2[:, :NNB]
    T2 = lax.bitcast_convert_type(P2[:, NNB:2 * NNB], jnp.float32)
    n2 = A2.reshape(-1)
    (E2,) = _sc_gather(n2, [emb])

    r2 = lambda x: x.reshape(1, -1)
    freq = (1.0 / (10.0 ** jnp.linspace(0.0, 9.0, TIME_DIM))).reshape(1, -1)
    h_top = _attn_stage(freq, E0, E1, cut.reshape(B, 1), T1, A1,
                        Wq[0], Wk[0], Wv[0], Wo[0],
                        r2(bq[0]), r2(bk[0]), r2(bv[0]), r2(bo[0]),
                        mW1[0], r2(mb1[0]), mW2[0], r2(mb2[0]))
    h_ngh = _attn_stage(freq, E1, E2, T1.reshape(-1, 1), T2, A2,
                        Wq[0], Wk[0], Wv[0], Wo[0],
                        r2(bq[0]), r2(bk[0]), r2(bv[0]), r2(bo[0]),
                        mW1[0], r2(mb1[0]), mW2[0], r2(mb2[0]))
    h_out = _attn_stage(freq, h_top, h_ngh, cut.reshape(B, 1), T1, A1,
                        Wq[1], Wk[1], Wv[1], Wo[1],
                        r2(bq[1]), r2(bk[1]), r2(bv[1]), r2(bo[1]),
                        mW1[1], r2(mb1[1]), mW2[1], r2(mb2[1]))
    half = B // 2
    score = _final_stage(h_out[:half], h_out[half:],
                         aW1, r2(ab1), aW2.reshape(1, -1), ab2.reshape(1, 1))
    return score[:, 0]


# speedup vs baseline: 1.2156x; 1.2156x over previous
import functools
import math

import jax
import jax.numpy as jnp
from jax import lax
from jax.experimental import pallas as pl
from jax.experimental.pallas import tpu as pltpu
from jax.experimental.pallas import tpu_sc as plsc

NODE_DIM = 128
TIME_DIM = 128
MODEL_DIM = NODE_DIM + TIME_DIM
N_HEAD = 4
HEAD_DIM = MODEL_DIM // N_HEAD
NNB = 20
_NW = 32


def _sc_gather(idx, tables):
    B = idx.shape[0]
    bpw = B // _NW
    C = min(bpw, 128)
    nch = bpw // C
    idx3 = idx.reshape(_NW, nch, C)
    nt = len(tables)
    mesh = plsc.VectorSubcoreMesh(core_axis_name="c", subcore_axis_name="s")
    out_type = tuple(
        jax.ShapeDtypeStruct((B, t.shape[1]), t.dtype) for t in tables
    )
    scratch = [pltpu.VMEM((nch, C), jnp.int32)]
    scratch += [pltpu.VMEM((C, t.shape[1]), t.dtype) for t in tables]
    scratch.append(pltpu.SemaphoreType.DMA)

    def body(*refs):
        tabs = refs[:nt]
        idx_ref = refs[nt]
        outs = refs[nt + 1:2 * nt + 1]
        idx_v = refs[2 * nt + 1]
        rows = refs[2 * nt + 2:2 * nt + 2 + nt]
        sem = refs[-1]
        wid = lax.axis_index("s") * 2 + lax.axis_index("c")
        pltpu.sync_copy(idx_ref.at[wid], idx_v)

        def step(i, carry):
            row0 = pl.multiple_of((wid * nch + i) * C, C)
            for t in range(nt):
                pltpu.async_copy(tabs[t].at[idx_v.at[i]], rows[t], sem).wait()
                pltpu.sync_copy(rows[t], outs[t].at[pl.ds(row0, C)])
            return carry

        lax.fori_loop(0, nch, step, 0)

    fn = pl.kernel(body, out_type=out_type, mesh=mesh, scratch_types=scratch)
    res = fn(*tables, idx3)
    return list(res)


def _attn_body(freq_r, eq_r, eng_r, dl_r, pen_r, wq_r, wk_r, wv_r,
               wo_r, bq_r, bk_r, bv_r, bo_r, mw1_r, mb1_r, mw2_r, mb2_r,
               out_r, *, QB):
    f32 = jnp.float32
    eq = eq_r[...]
    eng = eng_r[...]
    wq = wq_r[...]
    wk = wk_r[...]
    wv = wv_r[...]
    freq = freq_r[...]
    bf = jnp.bfloat16
    engb = eng.astype(bf)
    cq = jnp.sum(wq[NODE_DIM:, :], axis=0, keepdims=True) + bq_r[...]
    q = jnp.dot(eq.astype(bf), wq[:NODE_DIM, :].astype(bf),
                preferred_element_type=f32) + cq
    fs = freq * (1.0 / (2.0 * math.pi))
    u = jnp.dot(dl_r[...], fs, precision=lax.Precision.HIGHEST,
                preferred_element_type=f32)
    rr = u - jnp.floor(u + 0.5)
    w = rr * rr
    cosv = (((((-21.28321865 * w + 58.91265947) * w - 85.29598974) * w
              + 64.9306147) * w - 19.7390344) * w + 0.99999944)
    C = cosv.astype(bf)
    k = (jnp.dot(engb, wk[:NODE_DIM, :].astype(bf),
                 preferred_element_type=f32)
         + jnp.dot(C, wk[NODE_DIM:, :].astype(bf),
                   preferred_element_type=f32)
         + bk_r[...])
    v = (jnp.dot(engb, wv[:NODE_DIM, :].astype(bf),
                 preferred_element_type=f32)
         + jnp.dot(C, wv[NODE_DIM:, :].astype(bf),
                   preferred_element_type=f32)
         + bv_r[...])
    qb = jnp.broadcast_to(
        q[:, None, :], (QB, NNB, MODEL_DIM)).reshape(QB * NNB, MODEL_DIM)
    di = lax.broadcasted_iota(jnp.int32, (MODEL_DIM, N_HEAD), 0)
    hi = lax.broadcasted_iota(jnp.int32, (MODEL_DIM, N_HEAD), 1)
    R = (di // HEAD_DIM == hi).astype(bf)
    S2 = (jnp.dot((qb * k).astype(bf), R, preferred_element_type=f32)
          * (1.0 / 8.0) + pen_r[...])
    S = S2.reshape(QB, NNB, N_HEAD)
    m = jnp.max(S, axis=1, keepdims=True)
    Ex = jnp.exp(S - m)
    A = Ex / jnp.sum(Ex, axis=1, keepdims=True)
    Rt = (lax.broadcasted_iota(jnp.int32, (N_HEAD, MODEL_DIM), 1) // HEAD_DIM
          == lax.broadcasted_iota(jnp.int32, (N_HEAD, MODEL_DIM), 0)
          ).astype(bf)
    A4 = jnp.dot(A.reshape(QB * NNB, N_HEAD).astype(bf), Rt,
                 preferred_element_type=f32)
    o = jnp.sum((A4 * v).reshape(QB, NNB, MODEL_DIM), axis=1)
    o = (jnp.dot(o.astype(bf), wo_r[...].astype(bf),
                 preferred_element_type=f32) + bo_r[...])
    mw1 = mw1_r[...]
    g = (jnp.dot(o.astype(bf), mw1[:MODEL_DIM, :].astype(bf),
                 preferred_element_type=f32)
         + jnp.dot(eq.astype(bf), mw1[MODEL_DIM:, :].astype(bf),
                   preferred_element_type=f32)
         + mb1_r[...])
    g = jnp.maximum(g, 0.0)
    out_r[...] = (jnp.dot(g.astype(bf), mw2_r[...].astype(bf),
                          preferred_element_type=f32) + mb2_r[...])


def _attn_stage(freq, Eq, Eng, dl, pen, wq, wk, wv, wo, bq, bk, bv, bo,
                mw1, mb1, mw2, mb2, QB=128):
    Q = Eq.shape[0]
    grid = (Q // QB,)
    blk = lambda i: (i, 0)
    fix = lambda i: (0, 0)
    in_specs = [
        pl.BlockSpec((1, TIME_DIM), fix),
        pl.BlockSpec((QB, NODE_DIM), blk),
        pl.BlockSpec((QB * NNB, NODE_DIM), blk),
        pl.BlockSpec((QB * NNB, 1), blk),
        pl.BlockSpec((QB * NNB, 1), blk),
        pl.BlockSpec((MODEL_DIM, MODEL_DIM), fix),
        pl.BlockSpec((MODEL_DIM, MODEL_DIM), fix),
        pl.BlockSpec((MODEL_DIM, MODEL_DIM), fix),
        pl.BlockSpec((MODEL_DIM, MODEL_DIM), fix),
        pl.BlockSpec((1, MODEL_DIM), fix),
        pl.BlockSpec((1, MODEL_DIM), fix),
        pl.BlockSpec((1, MODEL_DIM), fix),
        pl.BlockSpec((1, MODEL_DIM), fix),
        pl.BlockSpec((MODEL_DIM + NODE_DIM, NODE_DIM), fix),
        pl.BlockSpec((1, NODE_DIM), fix),
        pl.BlockSpec((NODE_DIM, NODE_DIM), fix),
        pl.BlockSpec((1, NODE_DIM), fix),
    ]
    return pl.pallas_call(
        functools.partial(_attn_body, QB=QB),
        grid=grid,
        in_specs=in_specs,
        out_specs=pl.BlockSpec((QB, NODE_DIM), blk),
        out_shape=jax.ShapeDtypeStruct((Q, NODE_DIM), jnp.float32),
    )(freq, Eq, Eng, dl, pen, wq, wk, wv, wo, bq, bk, bv, bo,
      mw1, mb1, mw2, mb2)


def _final_body(s_r, t_r, aw1_r, ab1_r, aw2_r, ab2_r, out_r):
    f32 = jnp.float32
    aw1 = aw1_r[...]
    h = (jnp.dot(s_r[...], aw1[:NODE_DIM, :], preferred_element_type=f32)
         + jnp.dot(t_r[...], aw1[NODE_DIM:, :], preferred_element_type=f32)
         + ab1_r[...])
    h = jnp.maximum(h, 0.0)
    out_r[...] = jnp.sum(h * aw2_r[...], axis=1, keepdims=True) + ab2_r[...]


def _final_stage(src, tgt, aw1, ab1, aw2t, ab2):
    B = src.shape[0]
    return pl.pallas_call(
        _final_body,
        out_shape=jax.ShapeDtypeStruct((B, 1), jnp.float32),
    )(src, tgt, aw1, ab1, aw2t, ab2)


def kernel(src_idx_l, target_idx_l, cut_time_l, adj_nodes, adj_times, emb,
           Wq, Wk, Wv, Wo, bq, bk, bv, bo, mW1, mb1, mW2, mb2,
           aW1, ab1, aW2, ab2):
    idx = jnp.concatenate([src_idx_l, target_idx_l]).astype(jnp.int32)
    cut = jnp.concatenate([cut_time_l, cut_time_l])
    B = idx.shape[0]
    adj_i = adj_nodes.astype(jnp.int32)

    zpad = jnp.zeros((adj_i.shape[0], 128 - 2 * NNB), jnp.int32)
    packed = jnp.concatenate(
        [adj_i, lax.bitcast_convert_type(adj_times, jnp.int32), zpad], axis=1)

    P1, E0 = _sc_gather(idx, [packed, emb])
    A1 = P1[:, :NNB]
    T1 = lax.bitcast_convert_type(P1[:, NNB:2 * NNB], jnp.float32)
    n1 = A1.reshape(-1)
    P2, E1 = _sc_gather(n1, [packed, emb])
    A2 = P2[:, :NNB]
    T2 = lax.bitcast_convert_type(P2[:, NNB:2 * NNB], jnp.float32)
    n2 = A2.reshape(-1)
    (E2,) = _sc_gather(n2, [emb])

    r2 = lambda x: x.reshape(1, -1)
    freq = (1.0 / (10.0 ** jnp.linspace(0.0, 9.0, TIME_DIM))).reshape(1, -1)
    col = lambda x: x.reshape(-1, 1)
    dl1 = col(cut[:, None] - T1)
    dl2 = col(T1.reshape(-1)[:, None] - T2)
    pen1 = col(jnp.where(A1 == 0, -1e10, 0.0).astype(jnp.float32))
    pen2 = col(jnp.where(A2 == 0, -1e10, 0.0).astype(jnp.float32))
    h_top = _attn_stage(freq, E0, E1, dl1, pen1,
                        Wq[0], Wk[0], Wv[0], Wo[0],
                        r2(bq[0]), r2(bk[0]), r2(bv[0]), r2(bo[0]),
                        mW1[0], r2(mb1[0]), mW2[0], r2(mb2[0]))
    h_ngh = _attn_stage(freq, E1, E2, dl2, pen2,
                        Wq[0], Wk[0], Wv[0], Wo[0],
                        r2(bq[0]), r2(bk[0]), r2(bv[0]), r2(bo[0]),
                        mW1[0], r2(mb1[0]), mW2[0], r2(mb2[0]))
    h_out = _attn_stage(freq, h_top, h_ngh, dl1, pen1,
                        Wq[1], Wk[1], Wv[1], Wo[1],
                        r2(bq[1]), r2(bk[1]), r2(bv[1]), r2(bo[1]),
                        mW1[1], r2(mb1[1]), mW2[1], r2(mb2[1]))
    half = B // 2
    score = _final_stage(h_out[:half], h_out[half:],
                         aW1, r2(ab1), aW2.reshape(1, -1), ab2.reshape(1, 1))
    return score[:, 0]


# speedup vs baseline: 1.2505x; 1.0288x over previous
import functools
import math

import jax
import jax.numpy as jnp
from jax import lax
from jax.experimental import pallas as pl
from jax.experimental.pallas import tpu as pltpu
from jax.experimental.pallas import tpu_sc as plsc

NODE_DIM = 128
TIME_DIM = 128
MODEL_DIM = NODE_DIM + TIME_DIM
N_HEAD = 4
HEAD_DIM = MODEL_DIM // N_HEAD
NNB = 20
_NW = 32


def _sc_gather(idx, tables):
    B = idx.shape[0]
    bpw = B // _NW
    C = min(bpw, 128)
    nch = bpw // C
    idx3 = idx.reshape(_NW, nch, C)
    nt = len(tables)
    mesh = plsc.VectorSubcoreMesh(core_axis_name="c", subcore_axis_name="s")
    out_type = tuple(
        jax.ShapeDtypeStruct((B, t.shape[1]), t.dtype) for t in tables
    )
    scratch = [pltpu.VMEM((nch, C), jnp.int32)]
    scratch += [pltpu.VMEM((C, t.shape[1]), t.dtype) for t in tables]
    scratch.append(pltpu.SemaphoreType.DMA)

    def body(*refs):
        tabs = refs[:nt]
        idx_ref = refs[nt]
        outs = refs[nt + 1:2 * nt + 1]
        idx_v = refs[2 * nt + 1]
        rows = refs[2 * nt + 2:2 * nt + 2 + nt]
        sem = refs[-1]
        wid = lax.axis_index("s") * 2 + lax.axis_index("c")
        pltpu.sync_copy(idx_ref.at[wid], idx_v)

        def step(i, carry):
            row0 = pl.multiple_of((wid * nch + i) * C, C)
            for t in range(nt):
                pltpu.async_copy(tabs[t].at[idx_v.at[i]], rows[t], sem).wait()
                pltpu.sync_copy(rows[t], outs[t].at[pl.ds(row0, C)])
            return carry

        lax.fori_loop(0, nch, step, 0)

    fn = pl.kernel(body, out_type=out_type, mesh=mesh, scratch_types=scratch)
    res = fn(*tables, idx3)
    return list(res)


def _attn_body(freq_r, eq_r, eng_r, dl_r, pen_r, wq_r, wk_r, wv_r,
               wo_r, bq_r, bk_r, bv_r, bo_r, mw1_r, mb1_r, mw2_r, mb2_r,
               out_r, *, QB):
    f32 = jnp.float32
    eq = eq_r[...]
    eng = eng_r[...]
    wq = wq_r[...]
    wk = wk_r[...]
    wv = wv_r[...]
    freq = freq_r[...]
    bf = jnp.float32
    engb = eng.astype(bf)
    cq = jnp.sum(wq[NODE_DIM:, :], axis=0, keepdims=True) + bq_r[...]
    q = jnp.dot(eq.astype(bf), wq[:NODE_DIM, :].astype(bf),
                preferred_element_type=f32) + cq
    fs = freq * (1.0 / (2.0 * math.pi))
    ones_row = jnp.full((1, TIME_DIM), 1.0, f32)
    dbc = jnp.dot(dl_r[...], ones_row, precision=lax.Precision.HIGHEST,
                  preferred_element_type=f32)
    u = dbc * fs
    rr = u - jnp.floor(u + 0.5)
    w = rr * rr
    cosv = (((((-21.28321865 * w + 58.91265947) * w - 85.29598974) * w
              + 64.9306147) * w - 19.7390344) * w + 0.99999944)
    C = cosv.astype(bf)
    k = (jnp.dot(engb, wk[:NODE_DIM, :].astype(bf),
                 preferred_element_type=f32)
         + jnp.dot(C, wk[NODE_DIM:, :].astype(bf),
                   preferred_element_type=f32)
         + bk_r[...])
    v = (jnp.dot(engb, wv[:NODE_DIM, :].astype(bf),
                 preferred_element_type=f32)
         + jnp.dot(C, wv[NODE_DIM:, :].astype(bf),
                   preferred_element_type=f32)
         + bv_r[...])
    qb = jnp.broadcast_to(
        q[:, None, :], (QB, NNB, MODEL_DIM)).reshape(QB * NNB, MODEL_DIM)
    di = lax.broadcasted_iota(jnp.int32, (MODEL_DIM, N_HEAD), 0)
    hi = lax.broadcasted_iota(jnp.int32, (MODEL_DIM, N_HEAD), 1)
    R = (di // HEAD_DIM == hi).astype(bf)
    S2 = (jnp.dot((qb * k).astype(bf), R, preferred_element_type=f32)
          * (1.0 / 8.0) + pen_r[...])
    S = S2.reshape(QB, NNB, N_HEAD)
    m = jnp.max(S, axis=1, keepdims=True)
    Ex = jnp.exp(S - m)
    A = Ex / jnp.sum(Ex, axis=1, keepdims=True)
    Rt = (lax.broadcasted_iota(jnp.int32, (N_HEAD, MODEL_DIM), 1) // HEAD_DIM
          == lax.broadcasted_iota(jnp.int32, (N_HEAD, MODEL_DIM), 0)
          ).astype(bf)
    A4 = jnp.dot(A.reshape(QB * NNB, N_HEAD).astype(bf), Rt,
                 preferred_element_type=f32)
    o = jnp.sum((A4 * v).reshape(QB, NNB, MODEL_DIM), axis=1)
    o = (jnp.dot(o.astype(bf), wo_r[...].astype(bf),
                 preferred_element_type=f32) + bo_r[...])
    mw1 = mw1_r[...]
    g = (jnp.dot(o.astype(bf), mw1[:MODEL_DIM, :].astype(bf),
                 preferred_element_type=f32)
         + jnp.dot(eq.astype(bf), mw1[MODEL_DIM:, :].astype(bf),
                   preferred_element_type=f32)
         + mb1_r[...])
    g = jnp.maximum(g, 0.0)
    out_r[...] = (jnp.dot(g.astype(bf), mw2_r[...].astype(bf),
                          preferred_element_type=f32) + mb2_r[...])


def _attn_stage(freq, Eq, Eng, dl, pen, wq, wk, wv, wo, bq, bk, bv, bo,
                mw1, mb1, mw2, mb2, QB=128):
    Q = Eq.shape[0]
    grid = (Q // QB,)
    blk = lambda i: (i, 0)
    fix = lambda i: (0, 0)
    in_specs = [
        pl.BlockSpec((1, TIME_DIM), fix),
        pl.BlockSpec((QB, NODE_DIM), blk),
        pl.BlockSpec((QB * NNB, NODE_DIM), blk),
        pl.BlockSpec((QB * NNB, 1), blk),
        pl.BlockSpec((QB * NNB, 1), blk),
        pl.BlockSpec((MODEL_DIM, MODEL_DIM), fix),
        pl.BlockSpec((MODEL_DIM, MODEL_DIM), fix),
        pl.BlockSpec((MODEL_DIM, MODEL_DIM), fix),
        pl.BlockSpec((MODEL_DIM, MODEL_DIM), fix),
        pl.BlockSpec((1, MODEL_DIM), fix),
        pl.BlockSpec((1, MODEL_DIM), fix),
        pl.BlockSpec((1, MODEL_DIM), fix),
        pl.BlockSpec((1, MODEL_DIM), fix),
        pl.BlockSpec((MODEL_DIM + NODE_DIM, NODE_DIM), fix),
        pl.BlockSpec((1, NODE_DIM), fix),
        pl.BlockSpec((NODE_DIM, NODE_DIM), fix),
        pl.BlockSpec((1, NODE_DIM), fix),
    ]
    return pl.pallas_call(
        functools.partial(_attn_body, QB=QB),
        grid=grid,
        in_specs=in_specs,
        out_specs=pl.BlockSpec((QB, NODE_DIM), blk),
        out_shape=jax.ShapeDtypeStruct((Q, NODE_DIM), jnp.float32),
    )(freq, Eq, Eng, dl, pen, wq, wk, wv, wo, bq, bk, bv, bo,
      mw1, mb1, mw2, mb2)


def _final_body(s_r, t_r, aw1_r, ab1_r, aw2_r, ab2_r, out_r):
    f32 = jnp.float32
    aw1 = aw1_r[...]
    h = (jnp.dot(s_r[...], aw1[:NODE_DIM, :], preferred_element_type=f32)
         + jnp.dot(t_r[...], aw1[NODE_DIM:, :], preferred_element_type=f32)
         + ab1_r[...])
    h = jnp.maximum(h, 0.0)
    out_r[...] = jnp.sum(h * aw2_r[...], axis=1, keepdims=True) + ab2_r[...]


def _final_stage(src, tgt, aw1, ab1, aw2t, ab2):
    B = src.shape[0]
    return pl.pallas_call(
        _final_body,
        out_shape=jax.ShapeDtypeStruct((B, 1), jnp.float32),
    )(src, tgt, aw1, ab1, aw2t, ab2)


def kernel(src_idx_l, target_idx_l, cut_time_l, adj_nodes, adj_times, emb,
           Wq, Wk, Wv, Wo, bq, bk, bv, bo, mW1, mb1, mW2, mb2,
           aW1, ab1, aW2, ab2):
    idx = jnp.concatenate([src_idx_l, target_idx_l]).astype(jnp.int32)
    cut = jnp.concatenate([cut_time_l, cut_time_l])
    B = idx.shape[0]
    adj_i = adj_nodes.astype(jnp.int32)

    zpad = jnp.zeros((adj_i.shape[0], 128 - 2 * NNB), jnp.int32)
    packed = jnp.concatenate(
        [adj_i, lax.bitcast_convert_type(adj_times, jnp.int32), zpad], axis=1)

    P1, E0 = _sc_gather(idx, [packed, emb])
    A1 = P1[:, :NNB]
    T1 = lax.bitcast_convert_type(P1[:, NNB:2 * NNB], jnp.float32)
    n1 = A1.reshape(-1)
    P2, E1 = _sc_gather(n1, [packed, emb])
    A2 = P2[:, :NNB]
    T2 = lax.bitcast_convert_type(P2[:, NNB:2 * NNB], jnp.float32)
    n2 = A2.reshape(-1)
    (E2,) = _sc_gather(n2, [emb])

    r2 = lambda x: x.reshape(1, -1)
    freq = (1.0 / (10.0 ** jnp.linspace(0.0, 9.0, TIME_DIM))).reshape(1, -1)
    col = lambda x: x.reshape(-1, 1)
    dl1 = col(cut[:, None] - T1)
    dl2 = col(T1.reshape(-1)[:, None] - T2)
    pen1 = col(jnp.where(A1 == 0, -1e10, 0.0).astype(jnp.float32))
    pen2 = col(jnp.where(A2 == 0, -1e10, 0.0).astype(jnp.float32))
    h_top = _attn_stage(freq, E0, E1, dl1, pen1,
                        Wq[0], Wk[0], Wv[0], Wo[0],
                        r2(bq[0]), r2(bk[0]), r2(bv[0]), r2(bo[0]),
                        mW1[0], r2(mb1[0]), mW2[0], r2(mb2[0]))
    h_ngh = _attn_stage(freq, E1, E2, dl2, pen2,
                        Wq[0], Wk[0], Wv[0], Wo[0],
                        r2(bq[0]), r2(bk[0]), r2(bv[0]), r2(bo[0]),
                        mW1[0], r2(mb1[0]), mW2[0], r2(mb2[0]))
    h_out = _attn_stage(freq, h_top, h_ngh, dl1, pen1,
                        Wq[1], Wk[1], Wv[1], Wo[1],
                        r2(bq[1]), r2(bk[1]), r2(bv[1]), r2(bo[1]),
                        mW1[1], r2(mb1[1]), mW2[1], r2(mb2[1]))
    half = B // 2
    score = _final_stage(h_out[:half], h_out[half:],
                         aW1, r2(ab1), aW2.reshape(1, -1), ab2.reshape(1, 1))
    return score[:, 0]


# speedup vs baseline: 1.5428x; 1.2337x over previous
import functools
import math

import jax
import jax.numpy as jnp
from jax import lax
from jax.experimental import pallas as pl
from jax.experimental.pallas import tpu as pltpu
from jax.experimental.pallas import tpu_sc as plsc

NODE_DIM = 128
TIME_DIM = 128
MODEL_DIM = NODE_DIM + TIME_DIM
N_HEAD = 4
HEAD_DIM = MODEL_DIM // N_HEAD
NNB = 20
_NW = 32


def _sc_gather(idx, tables):
    B = idx.shape[0]
    bpw = B // _NW
    C = min(bpw, 128)
    nch = bpw // C
    idx3 = idx.reshape(_NW, nch, C)
    nt = len(tables)
    mesh = plsc.VectorSubcoreMesh(core_axis_name="c", subcore_axis_name="s")
    out_type = tuple(
        jax.ShapeDtypeStruct((B, t.shape[1]), t.dtype) for t in tables
    )
    scratch = [pltpu.VMEM((nch, C), jnp.int32)]
    scratch += [pltpu.VMEM((2, C, t.shape[1]), t.dtype) for t in tables]
    scratch.append(pltpu.SemaphoreType.DMA)
    scratch.append(pltpu.SemaphoreType.DMA)

    def body(*refs):
        tabs = refs[:nt]
        idx_ref = refs[nt]
        outs = refs[nt + 1:2 * nt + 1]
        idx_v = refs[2 * nt + 1]
        rows = refs[2 * nt + 2:2 * nt + 2 + nt]
        sems = refs[-2:]
        wid = lax.axis_index("s") * 2 + lax.axis_index("c")
        pltpu.sync_copy(idx_ref.at[wid], idx_v)

        def start(i, b):
            for t in range(nt):
                pltpu.make_async_copy(
                    tabs[t].at[idx_v.at[i]], rows[t].at[b], sems[b]).start()

        def drain(b):
            for t in range(nt):
                pltpu.make_async_copy(
                    tabs[t].at[idx_v.at[0]], rows[t].at[b], sems[b]).wait()

        def wout(i, b):
            row0 = pl.multiple_of((wid * nch + i) * C, C)
            for t in range(nt):
                pltpu.sync_copy(rows[t].at[b], outs[t].at[pl.ds(row0, C)])

        start(0, 0)

        def pair(j, carry):
            i0 = j * 2
            drain(0)
            start(i0 + 1, 1)
            wout(i0, 0)
            drain(1)

            @pl.when(i0 + 2 < nch)
            def _():
                start(i0 + 2, 0)

            wout(i0 + 1, 1)
            return carry

        lax.fori_loop(0, nch // 2, pair, 0)
        if nch % 2 == 1:
            drain(0)
            wout(nch - 1, 0)

    fn = pl.kernel(body, out_type=out_type, mesh=mesh, scratch_types=scratch)
    res = fn(*tables, idx3)
    return list(res)


def _attn_body(freq_r, eq_r, eng_r, dl_r, pen_r, wq_r, wk_r, wv_r,
               wo_r, bq_r, bk_r, bv_r, bo_r, mw1_r, mb1_r, mw2_r, mb2_r,
               out_r, *, QB):
    f32 = jnp.float32
    eq = eq_r[...]
    eng = eng_r[...]
    wq = wq_r[...]
    wk = wk_r[...]
    wv = wv_r[...]
    freq = freq_r[...]
    bf = jnp.float32
    engb = eng.astype(bf)
    cq = jnp.sum(wq[NODE_DIM:, :], axis=0, keepdims=True) + bq_r[...]
    q = jnp.dot(eq.astype(bf), wq[:NODE_DIM, :].astype(bf),
                preferred_element_type=f32) + cq
    fs = freq * (1.0 / (2.0 * math.pi))
    dbc = jnp.broadcast_to(dl_r[...], (QB * NNB, TIME_DIM))
    u = dbc * fs
    rr = u - jnp.floor(u + 0.5)
    w = rr * rr
    cosv = (((((-21.28321865 * w + 58.91265947) * w - 85.29598974) * w
              + 64.9306147) * w - 19.7390344) * w + 0.99999944)
    C = cosv.astype(bf)
    k = (jnp.dot(engb, wk[:NODE_DIM, :].astype(bf),
                 preferred_element_type=f32)
         + jnp.dot(C, wk[NODE_DIM:, :].astype(bf),
                   preferred_element_type=f32)
         + bk_r[...])
    v = (jnp.dot(engb, wv[:NODE_DIM, :].astype(bf),
                 preferred_element_type=f32)
         + jnp.dot(C, wv[NODE_DIM:, :].astype(bf),
                   preferred_element_type=f32)
         + bv_r[...])
    qb = jnp.broadcast_to(
        q[:, None, :], (QB, NNB, MODEL_DIM)).reshape(QB * NNB, MODEL_DIM)
    di = lax.broadcasted_iota(jnp.int32, (MODEL_DIM, N_HEAD), 0)
    hi = lax.broadcasted_iota(jnp.int32, (MODEL_DIM, N_HEAD), 1)
    R = (di // HEAD_DIM == hi).astype(bf)
    S2 = (jnp.dot((qb * k).astype(bf), R, preferred_element_type=f32)
          * (1.0 / 8.0) + pen_r[...])
    S = S2.reshape(QB, NNB, N_HEAD)
    m = jnp.max(S, axis=1, keepdims=True)
    Ex = jnp.exp(S - m)
    A = Ex / jnp.sum(Ex, axis=1, keepdims=True)
    Rt = (lax.broadcasted_iota(jnp.int32, (N_HEAD, MODEL_DIM), 1) // HEAD_DIM
          == lax.broadcasted_iota(jnp.int32, (N_HEAD, MODEL_DIM), 0)
          ).astype(bf)
    A4 = jnp.dot(A.reshape(QB * NNB, N_HEAD).astype(bf), Rt,
                 preferred_element_type=f32)
    o = jnp.sum((A4 * v).reshape(QB, NNB, MODEL_DIM), axis=1)
    o = (jnp.dot(o.astype(bf), wo_r[...].astype(bf),
                 preferred_element_type=f32) + bo_r[...])
    mw1 = mw1_r[...]
    g = (jnp.dot(o.astype(bf), mw1[:MODEL_DIM, :].astype(bf),
                 preferred_element_type=f32)
         + jnp.dot(eq.astype(bf), mw1[MODEL_DIM:, :].astype(bf),
                   preferred_element_type=f32)
         + mb1_r[...])
    g = jnp.maximum(g, 0.0)
    out_r[...] = (jnp.dot(g.astype(bf), mw2_r[...].astype(bf),
                          preferred_element_type=f32) + mb2_r[...])


def _attn_stage(freq, Eq, Eng, dl, pen, wq, wk, wv, wo, bq, bk, bv, bo,
                mw1, mb1, mw2, mb2, QB=128):
    Q = Eq.shape[0]
    grid = (Q // QB,)
    blk = lambda i: (i, 0)
    fix = lambda i: (0, 0)
    in_specs = [
        pl.BlockSpec((1, TIME_DIM), fix),
        pl.BlockSpec((QB, NODE_DIM), blk),
        pl.BlockSpec((QB * NNB, NODE_DIM), blk),
        pl.BlockSpec((QB * NNB, 1), blk),
        pl.BlockSpec((QB * NNB, 1), blk),
        pl.BlockSpec((MODEL_DIM, MODEL_DIM), fix),
        pl.BlockSpec((MODEL_DIM, MODEL_DIM), fix),
        pl.BlockSpec((MODEL_DIM, MODEL_DIM), fix),
        pl.BlockSpec((MODEL_DIM, MODEL_DIM), fix),
        pl.BlockSpec((1, MODEL_DIM), fix),
        pl.BlockSpec((1, MODEL_DIM), fix),
        pl.BlockSpec((1, MODEL_DIM), fix),
        pl.BlockSpec((1, MODEL_DIM), fix),
        pl.BlockSpec((MODEL_DIM + NODE_DIM, NODE_DIM), fix),
        pl.BlockSpec((1, NODE_DIM), fix),
        pl.BlockSpec((NODE_DIM, NODE_DIM), fix),
        pl.BlockSpec((1, NODE_DIM), fix),
    ]
    return pl.pallas_call(
        functools.partial(_attn_body, QB=QB),
        grid=grid,
        in_specs=in_specs,
        out_specs=pl.BlockSpec((QB, NODE_DIM), blk),
        out_shape=jax.ShapeDtypeStruct((Q, NODE_DIM), jnp.float32),
    )(freq, Eq, Eng, dl, pen, wq, wk, wv, wo, bq, bk, bv, bo,
      mw1, mb1, mw2, mb2)


def _final_body(s_r, t_r, aw1_r, ab1_r, aw2_r, ab2_r, out_r):
    f32 = jnp.float32
    aw1 = aw1_r[...]
    h = (jnp.dot(s_r[...], aw1[:NODE_DIM, :], preferred_element_type=f32)
         + jnp.dot(t_r[...], aw1[NODE_DIM:, :], preferred_element_type=f32)
         + ab1_r[...])
    h = jnp.maximum(h, 0.0)
    out_r[...] = jnp.sum(h * aw2_r[...], axis=1, keepdims=True) + ab2_r[...]


def _final_stage(src, tgt, aw1, ab1, aw2t, ab2):
    B = src.shape[0]
    return pl.pallas_call(
        _final_body,
        out_shape=jax.ShapeDtypeStruct((B, 1), jnp.float32),
    )(src, tgt, aw1, ab1, aw2t, ab2)


def kernel(src_idx_l, target_idx_l, cut_time_l, adj_nodes, adj_times, emb,
           Wq, Wk, Wv, Wo, bq, bk, bv, bo, mW1, mb1, mW2, mb2,
           aW1, ab1, aW2, ab2):
    idx = jnp.concatenate([src_idx_l, target_idx_l]).astype(jnp.int32)
    cut = jnp.concatenate([cut_time_l, cut_time_l])
    B = idx.shape[0]
    adj_i = adj_nodes.astype(jnp.int32)

    zpad = jnp.zeros((adj_i.shape[0], 128 - 2 * NNB), jnp.int32)
    packed = jnp.concatenate(
        [adj_i, lax.bitcast_convert_type(adj_times, jnp.int32), zpad], axis=1)

    P1, E0 = _sc_gather(idx, [packed, emb])
    A1 = P1[:, :NNB]
    T1 = lax.bitcast_convert_type(P1[:, NNB:2 * NNB], jnp.float32)
    n1 = A1.reshape(-1)
    P2, E1 = _sc_gather(n1, [packed, emb])
    A2 = P2[:, :NNB]
    T2 = lax.bitcast_convert_type(P2[:, NNB:2 * NNB], jnp.float32)
    n2 = A2.reshape(-1)
    (E2,) = _sc_gather(n2, [emb])

    r2 = lambda x: x.reshape(1, -1)
    freq = (1.0 / (10.0 ** jnp.linspace(0.0, 9.0, TIME_DIM))).reshape(1, -1)
    col = lambda x: x.reshape(-1, 1)
    dl1 = col(cut[:, None] - T1)
    dl2 = col(T1.reshape(-1)[:, None] - T2)
    pen1 = col(jnp.where(A1 == 0, -1e10, 0.0).astype(jnp.float32))
    pen2 = col(jnp.where(A2 == 0, -1e10, 0.0).astype(jnp.float32))
    h_top = _attn_stage(freq, E0, E1, dl1, pen1,
                        Wq[0], Wk[0], Wv[0], Wo[0],
                        r2(bq[0]), r2(bk[0]), r2(bv[0]), r2(bo[0]),
                        mW1[0], r2(mb1[0]), mW2[0], r2(mb2[0]))
    h_ngh = _attn_stage(freq, E1, E2, dl2, pen2,
                        Wq[0], Wk[0], Wv[0], Wo[0],
                        r2(bq[0]), r2(bk[0]), r2(bv[0]), r2(bo[0]),
                        mW1[0], r2(mb1[0]), mW2[0], r2(mb2[0]))
    h_out = _attn_stage(freq, h_top, h_ngh, dl1, pen1,
                        Wq[1], Wk[1], Wv[1], Wo[1],
                        r2(bq[1]), r2(bk[1]), r2(bv[1]), r2(bo[1]),
                        mW1[1], r2(mb1[1]), mW2[1], r2(mb2[1]))
    half = B // 2
    score = _final_stage(h_out[:half], h_out[half:],
                         aW1, r2(ab1), aW2.reshape(1, -1), ab2.reshape(1, 1))
    return score[:, 0]


# speedup vs baseline: 1.5842x; 1.0268x over previous
import functools
import math

import jax
import jax.numpy as jnp
from jax import lax
from jax.experimental import pallas as pl
from jax.experimental.pallas import tpu as pltpu
from jax.experimental.pallas import tpu_sc as plsc

NODE_DIM = 128
TIME_DIM = 128
MODEL_DIM = NODE_DIM + TIME_DIM
N_HEAD = 4
HEAD_DIM = MODEL_DIM // N_HEAD
NNB = 20
_NW = 32


def _sc_gather(idx, tables):
    B = idx.shape[0]
    bpw = B // _NW
    C = min(bpw, 128)
    nch = bpw // C
    idx3 = idx.reshape(_NW, nch, C)
    nt = len(tables)
    mesh = plsc.VectorSubcoreMesh(core_axis_name="c", subcore_axis_name="s")
    out_type = tuple(
        jax.ShapeDtypeStruct((B, t.shape[1]), t.dtype) for t in tables
    )
    scratch = [pltpu.VMEM((nch, C), jnp.int32)]
    scratch += [pltpu.VMEM((2, C, t.shape[1]), t.dtype) for t in tables]
    scratch.append(pltpu.SemaphoreType.DMA)
    scratch.append(pltpu.SemaphoreType.DMA)

    def body(*refs):
        tabs = refs[:nt]
        idx_ref = refs[nt]
        outs = refs[nt + 1:2 * nt + 1]
        idx_v = refs[2 * nt + 1]
        rows = refs[2 * nt + 2:2 * nt + 2 + nt]
        sems = refs[-2:]
        wid = lax.axis_index("s") * 2 + lax.axis_index("c")
        pltpu.sync_copy(idx_ref.at[wid], idx_v)

        def start(i, b):
            for t in range(nt):
                pltpu.make_async_copy(
                    tabs[t].at[idx_v.at[i]], rows[t].at[b], sems[b]).start()

        def drain(b):
            for t in range(nt):
                pltpu.make_async_copy(
                    tabs[t].at[idx_v.at[0]], rows[t].at[b], sems[b]).wait()

        def wout(i, b):
            row0 = pl.multiple_of((wid * nch + i) * C, C)
            for t in range(nt):
                pltpu.sync_copy(rows[t].at[b], outs[t].at[pl.ds(row0, C)])

        start(0, 0)

        def pair(j, carry):
            i0 = j * 2
            drain(0)
            start(i0 + 1, 1)
            wout(i0, 0)
            drain(1)

            @pl.when(i0 + 2 < nch)
            def _():
                start(i0 + 2, 0)

            wout(i0 + 1, 1)
            return carry

        lax.fori_loop(0, nch // 2, pair, 0)
        if nch % 2 == 1:
            drain(0)
            wout(nch - 1, 0)

    fn = pl.kernel(body, out_type=out_type, mesh=mesh, scratch_types=scratch)
    res = fn(*tables, idx3)
    return list(res)


def _attn_body(freq_r, eq_r, eng_r, dl_r, pen_r, wq_r, wk_r, wv_r,
               wo_r, bq_r, bk_r, bv_r, bo_r, mw1_r, mb1_r, mw2_r, mb2_r,
               out_r, *, QB):
    f32 = jnp.float32
    eq = eq_r[...]
    eng = eng_r[...]
    wq = wq_r[...]
    wk = wk_r[...]
    wv = wv_r[...]
    freq = freq_r[...]
    bf = jnp.float32
    engb = eng.astype(bf)
    cq = jnp.sum(wq[NODE_DIM:, :], axis=0, keepdims=True) + bq_r[...]
    q = jnp.dot(eq.astype(bf), wq[:NODE_DIM, :].astype(bf),
                preferred_element_type=f32) + cq
    fs = freq * (1.0 / (2.0 * math.pi))
    dbc = jnp.broadcast_to(dl_r[...], (QB * NNB, TIME_DIM))
    u = dbc * fs
    rr = u - jnp.floor(u + 0.5)
    w = rr * rr
    cosv = (((((-21.28321865 * w + 58.91265947) * w - 85.29598974) * w
              + 64.9306147) * w - 19.7390344) * w + 0.99999944)
    C = cosv.astype(bf)
    k = (jnp.dot(engb, wk[:NODE_DIM, :].astype(bf),
                 preferred_element_type=f32)
         + jnp.dot(C, wk[NODE_DIM:, :].astype(bf),
                   preferred_element_type=f32)
         + bk_r[...])
    v = (jnp.dot(engb, wv[:NODE_DIM, :].astype(bf),
                 preferred_element_type=f32)
         + jnp.dot(C, wv[NODE_DIM:, :].astype(bf),
                   preferred_element_type=f32)
         + bv_r[...])
    qb = jnp.broadcast_to(
        q[:, None, :], (QB, NNB, MODEL_DIM)).reshape(QB * NNB, MODEL_DIM)
    di = lax.broadcasted_iota(jnp.int32, (MODEL_DIM, N_HEAD), 0)
    hi = lax.broadcasted_iota(jnp.int32, (MODEL_DIM, N_HEAD), 1)
    R = (di // HEAD_DIM == hi).astype(bf)
    S2 = (jnp.dot((qb * k).astype(bf), R, preferred_element_type=f32)
          * (1.0 / 8.0) + pen_r[...])
    S = S2.reshape(QB, NNB, N_HEAD)
    m = jnp.max(S, axis=1, keepdims=True)
    Ex = jnp.exp(S - m)
    A = Ex / jnp.sum(Ex, axis=1, keepdims=True)
    Rt = (lax.broadcasted_iota(jnp.int32, (N_HEAD, MODEL_DIM), 1) // HEAD_DIM
          == lax.broadcasted_iota(jnp.int32, (N_HEAD, MODEL_DIM), 0)
          ).astype(bf)
    A4 = jnp.dot(A.reshape(QB * NNB, N_HEAD).astype(bf), Rt,
                 preferred_element_type=f32)
    o = jnp.sum((A4 * v).reshape(QB, NNB, MODEL_DIM), axis=1)
    o = (jnp.dot(o.astype(bf), wo_r[...].astype(bf),
                 preferred_element_type=f32) + bo_r[...])
    mw1 = mw1_r[...]
    g = (jnp.dot(o.astype(bf), mw1[:MODEL_DIM, :].astype(bf),
                 preferred_element_type=f32)
         + jnp.dot(eq.astype(bf), mw1[MODEL_DIM:, :].astype(bf),
                   preferred_element_type=f32)
         + mb1_r[...])
    g = jnp.maximum(g, 0.0)
    out_r[...] = (jnp.dot(g.astype(bf), mw2_r[...].astype(bf),
                          preferred_element_type=f32) + mb2_r[...])


def _attn_stage(freq, Eq, Eng, dl, pen, wq, wk, wv, wo, bq, bk, bv, bo,
                mw1, mb1, mw2, mb2, QB=256):
    Q = Eq.shape[0]
    grid = (Q // QB,)
    blk = lambda i: (i, 0)
    fix = lambda i: (0, 0)
    in_specs = [
        pl.BlockSpec((1, TIME_DIM), fix),
        pl.BlockSpec((QB, NODE_DIM), blk),
        pl.BlockSpec((QB * NNB, NODE_DIM), blk),
        pl.BlockSpec((QB * NNB, 1), blk),
        pl.BlockSpec((QB * NNB, 1), blk),
        pl.BlockSpec((MODEL_DIM, MODEL_DIM), fix),
        pl.BlockSpec((MODEL_DIM, MODEL_DIM), fix),
        pl.BlockSpec((MODEL_DIM, MODEL_DIM), fix),
        pl.BlockSpec((MODEL_DIM, MODEL_DIM), fix),
        pl.BlockSpec((1, MODEL_DIM), fix),
        pl.BlockSpec((1, MODEL_DIM), fix),
        pl.BlockSpec((1, MODEL_DIM), fix),
        pl.BlockSpec((1, MODEL_DIM), fix),
        pl.BlockSpec((MODEL_DIM + NODE_DIM, NODE_DIM), fix),
        pl.BlockSpec((1, NODE_DIM), fix),
        pl.BlockSpec((NODE_DIM, NODE_DIM), fix),
        pl.BlockSpec((1, NODE_DIM), fix),
    ]
    return pl.pallas_call(
        functools.partial(_attn_body, QB=QB),
        grid=grid,
        in_specs=in_specs,
        out_specs=pl.BlockSpec((QB, NODE_DIM), blk),
        out_shape=jax.ShapeDtypeStruct((Q, NODE_DIM), jnp.float32),
    )(freq, Eq, Eng, dl, pen, wq, wk, wv, wo, bq, bk, bv, bo,
      mw1, mb1, mw2, mb2)


def _final_body(s_r, t_r, aw1_r, ab1_r, aw2_r, ab2_r, out_r):
    f32 = jnp.float32
    aw1 = aw1_r[...]
    h = (jnp.dot(s_r[...], aw1[:NODE_DIM, :], preferred_element_type=f32)
         + jnp.dot(t_r[...], aw1[NODE_DIM:, :], preferred_element_type=f32)
         + ab1_r[...])
    h = jnp.maximum(h, 0.0)
    out_r[...] = jnp.sum(h * aw2_r[...], axis=1, keepdims=True) + ab2_r[...]


def _final_stage(src, tgt, aw1, ab1, aw2t, ab2):
    B = src.shape[0]
    return pl.pallas_call(
        _final_body,
        out_shape=jax.ShapeDtypeStruct((B, 1), jnp.float32),
    )(src, tgt, aw1, ab1, aw2t, ab2)


def kernel(src_idx_l, target_idx_l, cut_time_l, adj_nodes, adj_times, emb,
           Wq, Wk, Wv, Wo, bq, bk, bv, bo, mW1, mb1, mW2, mb2,
           aW1, ab1, aW2, ab2):
    idx = jnp.concatenate([src_idx_l, target_idx_l]).astype(jnp.int32)
    cut = jnp.concatenate([cut_time_l, cut_time_l])
    B = idx.shape[0]
    adj_i = adj_nodes.astype(jnp.int32)

    zpad = jnp.zeros((adj_i.shape[0], 128 - 2 * NNB), jnp.int32)
    packed = jnp.concatenate(
        [adj_i, lax.bitcast_convert_type(adj_times, jnp.int32), zpad], axis=1)

    P1, E0 = _sc_gather(idx, [packed, emb])
    A1 = P1[:, :NNB]
    T1 = lax.bitcast_convert_type(P1[:, NNB:2 * NNB], jnp.float32)
    n1 = A1.reshape(-1)
    P2, E1 = _sc_gather(n1, [packed, emb])
    A2 = P2[:, :NNB]
    T2 = lax.bitcast_convert_type(P2[:, NNB:2 * NNB], jnp.float32)
    n2 = A2.reshape(-1)
    (E2,) = _sc_gather(n2, [emb])

    r2 = lambda x: x.reshape(1, -1)
    freq = (1.0 / (10.0 ** jnp.linspace(0.0, 9.0, TIME_DIM))).reshape(1, -1)
    col = lambda x: x.reshape(-1, 1)
    dl1 = col(cut[:, None] - T1)
    dl2 = col(T1.reshape(-1)[:, None] - T2)
    pen1 = col(jnp.where(A1 == 0, -1e10, 0.0).astype(jnp.float32))
    pen2 = col(jnp.where(A2 == 0, -1e10, 0.0).astype(jnp.float32))
    h_top = _attn_stage(freq, E0, E1, dl1, pen1,
                        Wq[0], Wk[0], Wv[0], Wo[0],
                        r2(bq[0]), r2(bk[0]), r2(bv[0]), r2(bo[0]),
                        mW1[0], r2(mb1[0]), mW2[0], r2(mb2[0]))
    h_ngh = _attn_stage(freq, E1, E2, dl2, pen2,
                        Wq[0], Wk[0], Wv[0], Wo[0],
                        r2(bq[0]), r2(bk[0]), r2(bv[0]), r2(bo[0]),
                        mW1[0], r2(mb1[0]), mW2[0], r2(mb2[0]))
    h_out = _attn_stage(freq, h_top, h_ngh, dl1, pen1,
                        Wq[1], Wk[1], Wv[1], Wo[1],
                        r2(bq[1]), r2(bk[1]), r2(bv[1]), r2(bo[1]),
                        mW1[1], r2(mb1[1]), mW2[1], r2(mb2[1]))
    half = B // 2
    score = _final_stage(h_out[:half], h_out[half:],
                         aW1, r2(ab1), aW2.reshape(1, -1), ab2.reshape(1, 1))
    return score[:, 0]
